# Initial kernel scaffold; baseline (speedup 1.0000x reference)
import jax
import jax.numpy as jnp
from jax.experimental import pallas as pl


def kernel(pred, target):
    raise NotImplementedError("write your pallas kernel here")


# speedup vs baseline: 25.9220x; 25.9220x over previous
import jax
import jax.numpy as jnp
from jax.experimental import pallas as pl

_C = 1000
_RATIO = 3
_ROWS_PER_BLOCK = 256


def _softplus(x):
    return jnp.maximum(x, 0.0) + jnp.log1p(jnp.exp(-jnp.abs(x)))


def _block_kernel(pred_ref, target_ref, num_ref, den_ref):
    x = pred_ref[...]
    y = target_ref[...]
    s = _softplus(x)
    contrib = s - y * x
    pos = jnp.sum(y, axis=1)
    full = jnp.sum(contrib, axis=1)
    d_f = jnp.maximum(_C - (_RATIO + 1.0) * pos, 0.0)

    @pl.when(pl.program_id(0) == 0)
    def _init():
        num_ref[...] = jnp.zeros((1, 1), jnp.float32)
        den_ref[...] = jnp.zeros((1, 1), jnp.float32)

    num_ref[...] += jnp.sum(full).reshape(1, 1)
    den_ref[...] += jnp.sum(pos).reshape(1, 1)

    @pl.when(jnp.any(d_f > 0.0))
    def _rare_correction():
        b = jax.lax.bitcast_convert_type(x, jnp.int32)
        ikey = b ^ ((b >> 31) & jnp.int32(0x7FFFFFFF))
        ikey = jnp.where(y > 0.5, jnp.int32(0x7FFFFFFF), ikey)
        d = d_f.astype(jnp.int32)
        cnt_neg = jnp.sum((ikey < 0).astype(jnp.int32), axis=1)
        t0 = jnp.where(cnt_neg >= d, jnp.int32(-2147483648), jnp.int32(0))

        def body(i, t):
            cand = t + (jnp.int32(1) << (30 - i))
            cnt = jnp.sum((ikey < cand[:, None]).astype(jnp.int32), axis=1)
            return jnp.where(cnt < d, cand, t)

        t = jax.lax.fori_loop(0, 31, body, t0)
        below = ikey < t[:, None]
        cnt_lt = jnp.sum(below.astype(jnp.int32), axis=1)
        sum_below = jnp.sum(jnp.where(below, s, 0.0), axis=1)
        bv = t ^ ((t >> 31) & jnp.int32(0x7FFFFFFF))
        sv = _softplus(jax.lax.bitcast_convert_type(bv, jnp.float32))
        corr = sum_below + (d - cnt_lt).astype(jnp.float32) * sv
        corr = jnp.where(d > 0, corr, 0.0)
        num_ref[...] += -jnp.sum(corr).reshape(1, 1)


def kernel(pred, target):
    n, c = pred.shape
    r = _ROWS_PER_BLOCK
    num, den = pl.pallas_call(
        _block_kernel,
        grid=(n // r,),
        in_specs=[
            pl.BlockSpec((r, c), lambda i: (i, 0)),
            pl.BlockSpec((r, c), lambda i: (i, 0)),
        ],
        out_specs=[
            pl.BlockSpec((1, 1), lambda i: (0, 0)),
            pl.BlockSpec((1, 1), lambda i: (0, 0)),
        ],
        out_shape=[
            jax.ShapeDtypeStruct((1, 1), jnp.float32),
            jax.ShapeDtypeStruct((1, 1), jnp.float32),
        ],
    )(pred, target)
    return (num[0, 0] / c) / den[0, 0]
